# baseline (speedup 1.0000x reference)
import functools

import jax
import jax.numpy as jnp
from jax import lax
from jax.experimental import pallas as pl
from jax.experimental.pallas import tpu as pltpu
from jax.experimental.pallas import tpu_sc as plsc

VOCAB = 100000
EMBED = 64
B = 1024
L = 50

NC = 2
NS = 16
NW = NC * NS
DPW = EMBED // NW

CB = 128
VB = 4096


def _poolt_body(cwt_hbm, embt_hbm, out_hbm, row_v, cw0_v, cw1_v, acc_v,
                sem0, sem1):
    wid = lax.axis_index("s") * NC + lax.axis_index("c")
    nbc = B // CB
    cwb = (cw0_v, cw1_v)
    sems = (sem0, sem1)

    def fire_cw(k):
        bc = k % nbc
        return pltpu.async_copy(
            cwt_hbm.at[:, pl.ds(bc * CB, CB)], cwb[k % 2], sems[k % 2])

    cw_cp = [None] * (DPW * nbc)
    cw_cp[0] = fire_cw(0)
    for dl in range(DPW):
        d = DPW * wid + dl
        pltpu.sync_copy(embt_hbm.at[d], row_v)
        for bc in range(nbc):
            k = dl * nbc + bc
            cw_cp[k].wait()
            if k + 1 < DPW * nbc:
                cw_cp[k + 1] = fire_cw(k + 1)
            cw_v = cwb[k % 2]

            @plsc.parallel_loop(0, CB // 16, 1, unroll=1)
            def gbody(g, cw_v=cw_v, dl=dl, bc=bc):
                acc = jnp.zeros((16,), jnp.float32)
                for i in range(L):
                    idx = cw_v[i, pl.ds(16 * g, 16)]
                    acc = acc + plsc.load_gather(row_v, [idx])
                acc_v[dl, pl.ds(bc * CB + 16 * g, 16)] = acc

    pltpu.sync_copy(acc_v, out_hbm.at[pl.ds(DPW * wid, DPW)])


@jax.jit
def _poolt(cwt, embt):
    return pl.kernel(
        _poolt_body,
        out_type=jax.ShapeDtypeStruct((EMBED, B), jnp.float32),
        mesh=plsc.VectorSubcoreMesh(core_axis_name="c", subcore_axis_name="s"),
        scratch_types=[
            pltpu.VMEM((VOCAB,), jnp.float32),
            pltpu.VMEM((L, CB), jnp.int32),
            pltpu.VMEM((L, CB), jnp.int32),
            pltpu.VMEM((DPW, B), jnp.float32),
            pltpu.SemaphoreType.DMA,
            pltpu.SemaphoreType.DMA,
        ],
        compiler_params=pltpu.CompilerParams(
            use_tc_tiling_on_sc=True, needs_layout_passes=False),
    )(cwt, embt)


def _proj_body(xt_ref, wt_ref, b_ref, outt_ref, xn_ref):
    @pl.when(pl.program_id(0) == 0)
    def _():
        xt = xt_ref[...]
        mu = jnp.mean(xt, axis=1, keepdims=True)
        xc = xt - mu
        var = jnp.mean(xc * xc, axis=1, keepdims=True)
        xn_ref[...] = xc * lax.rsqrt(var + 1e-10)

    acc = lax.dot_general(
        wt_ref[...], xn_ref[...],
        (((0,), (0,)), ((), ())),
        preferred_element_type=jnp.float32,
    )
    bias = lax.dot_general(
        b_ref[...], jnp.ones((1, B), jnp.float32),
        (((0,), (0,)), ((), ())),
        preferred_element_type=jnp.float32,
    )
    outt_ref[...] = acc + bias


@jax.jit
def _proj(xt, wt, b2d):
    grid = (pl.cdiv(VOCAB, VB),)
    return pl.pallas_call(
        _proj_body,
        grid=grid,
        in_specs=[
            pl.BlockSpec((EMBED, B), lambda i: (0, 0)),
            pl.BlockSpec((EMBED, VB), lambda i: (0, i)),
            pl.BlockSpec((1, VB), lambda i: (0, i)),
        ],
        out_specs=pl.BlockSpec((VB, B), lambda i: (i, 0)),
        out_shape=jax.ShapeDtypeStruct((VOCAB, B), jnp.float32),
        scratch_shapes=[pltpu.VMEM((EMBED, B), jnp.float32)],
    )(xt, wt, b2d)


def kernel(context_words, emb, W, b):
    cwt = context_words.astype(jnp.int32).T
    xt = _poolt(cwt, emb.T)
    outt = _proj(xt, W.T, b.reshape(1, VOCAB))
    return outt.T


# speedup vs baseline: 1.0099x; 1.0099x over previous
import functools

import jax
import jax.numpy as jnp
from jax import lax
from jax.experimental import pallas as pl
from jax.experimental.pallas import tpu as pltpu
from jax.experimental.pallas import tpu_sc as plsc

VOCAB = 100000
EMBED = 64
B = 1024
L = 50

NC = 2
NS = 16
NW = NC * NS
DPW = EMBED // NW

CB = 128
VB = 4096


def _poolt_body(cwt_hbm, embt_hbm, out_hbm, row_v, cw0_v, cw1_v, acc_v,
                sem0, sem1):
    wid = lax.axis_index("s") * NC + lax.axis_index("c")
    nbc = B // CB
    cwb = (cw0_v, cw1_v)
    sems = (sem0, sem1)

    def fire_cw(k):
        bc = k % nbc
        return pltpu.async_copy(
            cwt_hbm.at[:, pl.ds(bc * CB, CB)], cwb[k % 2], sems[k % 2])

    cw_cp = [None] * (DPW * nbc)
    cw_cp[0] = fire_cw(0)
    for dl in range(DPW):
        d = DPW * wid + dl
        pltpu.sync_copy(embt_hbm.at[d], row_v)
        for bc in range(nbc):
            k = dl * nbc + bc
            cw_cp[k].wait()
            if k + 1 < DPW * nbc:
                cw_cp[k + 1] = fire_cw(k + 1)
            cw_v = cwb[k % 2]

            @plsc.parallel_loop(0, CB // 16, 1, unroll=1)
            def gbody(g, cw_v=cw_v, dl=dl, bc=bc):
                accs = [jnp.zeros((16,), jnp.float32) for _ in range(4)]
                for i in range(L):
                    idx = cw_v[i, pl.ds(16 * g, 16)]
                    accs[i % 4] = accs[i % 4] + plsc.load_gather(row_v, [idx])
                acc = (accs[0] + accs[1]) + (accs[2] + accs[3])
                acc_v[dl, pl.ds(bc * CB + 16 * g, 16)] = acc

    pltpu.sync_copy(acc_v, out_hbm.at[pl.ds(DPW * wid, DPW)])


@jax.jit
def _poolt(cwt, embt):
    return pl.kernel(
        _poolt_body,
        out_type=jax.ShapeDtypeStruct((EMBED, B), jnp.float32),
        mesh=plsc.VectorSubcoreMesh(core_axis_name="c", subcore_axis_name="s"),
        scratch_types=[
            pltpu.VMEM((VOCAB,), jnp.float32),
            pltpu.VMEM((L, CB), jnp.int32),
            pltpu.VMEM((L, CB), jnp.int32),
            pltpu.VMEM((DPW, B), jnp.float32),
            pltpu.SemaphoreType.DMA,
            pltpu.SemaphoreType.DMA,
        ],
        compiler_params=pltpu.CompilerParams(
            use_tc_tiling_on_sc=True, needs_layout_passes=False),
    )(cwt, embt)


def _proj_body(xt_ref, wt_ref, b_ref, outt_ref, xn_ref):
    @pl.when(pl.program_id(0) == 0)
    def _():
        xt = xt_ref[...]
        mu = jnp.mean(xt, axis=1, keepdims=True)
        xc = xt - mu
        var = jnp.mean(xc * xc, axis=1, keepdims=True)
        xn_ref[...] = xc * lax.rsqrt(var + 1e-10)

    acc = lax.dot_general(
        wt_ref[...], xn_ref[...],
        (((0,), (0,)), ((), ())),
        preferred_element_type=jnp.float32,
    )
    bias = lax.dot_general(
        b_ref[...], jnp.ones((1, B), jnp.float32),
        (((0,), (0,)), ((), ())),
        preferred_element_type=jnp.float32,
    )
    outt_ref[...] = acc + bias


@jax.jit
def _proj(xt, wt, b2d):
    grid = (pl.cdiv(VOCAB, VB),)
    return pl.pallas_call(
        _proj_body,
        grid=grid,
        in_specs=[
            pl.BlockSpec((EMBED, B), lambda i: (0, 0)),
            pl.BlockSpec((EMBED, VB), lambda i: (0, i)),
            pl.BlockSpec((1, VB), lambda i: (0, i)),
        ],
        out_specs=pl.BlockSpec((VB, B), lambda i: (i, 0)),
        out_shape=jax.ShapeDtypeStruct((VOCAB, B), jnp.float32),
        scratch_shapes=[pltpu.VMEM((EMBED, B), jnp.float32)],
    )(xt, wt, b2d)


def kernel(context_words, emb, W, b):
    cwt = context_words.astype(jnp.int32).T
    xt = _poolt(cwt, emb.T)
    outt = _proj(xt, W.T, b.reshape(1, VOCAB))
    return outt.T


# speedup vs baseline: 1.0101x; 1.0003x over previous
import jax
import jax.numpy as jnp
from jax import lax
from jax.experimental import pallas as pl
from jax.experimental.pallas import tpu as pltpu
from jax.experimental.pallas import tpu_sc as plsc

VOCAB = 100000
EMBED = 64
B = 1024
L = 50

NC = 2
NS = 16
NW = NC * NS
DPW = EMBED // NW

CB = 128
VB = 4096


def _poolt_body(cwt_hbm, embt_hbm, out_hbm, row_v, cw0_v, cw1_v, acc_v,
                sem0, sem1):
    wid = lax.axis_index("s") * NC + lax.axis_index("c")
    nbc = B // CB
    cwb = (cw0_v, cw1_v)
    sems = (sem0, sem1)

    def fire_cw(k):
        bc = k % nbc
        return pltpu.async_copy(
            cwt_hbm.at[:, pl.ds(bc * CB, CB)], cwb[k % 2], sems[k % 2])

    cw_cp = [None] * (DPW * nbc)
    cw_cp[0] = fire_cw(0)
    for dl in range(DPW):
        d = DPW * wid + dl
        pltpu.sync_copy(embt_hbm.at[d], row_v)
        for bc in range(nbc):
            k = dl * nbc + bc
            cw_cp[k].wait()
            if k + 1 < DPW * nbc:
                cw_cp[k + 1] = fire_cw(k + 1)
            cw_v = cwb[k % 2]

            @plsc.parallel_loop(0, CB // 16, 1, unroll=1)
            def gbody(g, cw_v=cw_v, dl=dl, bc=bc):
                accs = [jnp.zeros((16,), jnp.float32) for _ in range(4)]
                for i in range(L):
                    idx = cw_v[i, pl.ds(16 * g, 16)]
                    accs[i % 4] = accs[i % 4] + plsc.load_gather(row_v, [idx])
                acc = (accs[0] + accs[1]) + (accs[2] + accs[3])
                acc_v[dl, pl.ds(bc * CB + 16 * g, 16)] = acc

    pltpu.sync_copy(acc_v, out_hbm.at[pl.ds(DPW * wid, DPW)])


@jax.jit
def _poolt(cwt, embt):
    return pl.kernel(
        _poolt_body,
        out_type=jax.ShapeDtypeStruct((EMBED, B), jnp.float32),
        mesh=plsc.VectorSubcoreMesh(core_axis_name="c", subcore_axis_name="s"),
        scratch_types=[
            pltpu.VMEM((VOCAB,), jnp.float32),
            pltpu.VMEM((L, CB), jnp.int32),
            pltpu.VMEM((L, CB), jnp.int32),
            pltpu.VMEM((DPW, B), jnp.float32),
            pltpu.SemaphoreType.DMA,
            pltpu.SemaphoreType.DMA,
        ],
        compiler_params=pltpu.CompilerParams(
            use_tc_tiling_on_sc=True, needs_layout_passes=False),
    )(cwt, embt)


def _proj_body(xt_ref, wt_ref, b_ref, outt_ref, xn_ref):
    @pl.when(pl.program_id(0) == 0)
    def _():
        xt = xt_ref[...]
        mu = jnp.mean(xt, axis=1, keepdims=True)
        xc = xt - mu
        var = jnp.mean(xc * xc, axis=1, keepdims=True)
        xn_ref[...] = xc * lax.rsqrt(var + 1e-10)

    acc = lax.dot_general(
        wt_ref[...], xn_ref[...],
        (((0,), (0,)), ((), ())),
        preferred_element_type=jnp.float32,
    )
    bias = lax.dot_general(
        b_ref[...], jnp.ones((1, B), jnp.float32),
        (((0,), (0,)), ((), ())),
        preferred_element_type=jnp.float32,
    )
    outt_ref[...] = acc + bias


@jax.jit
def _proj(xt, wt, b2d):
    grid = (pl.cdiv(VOCAB, VB),)
    return pl.pallas_call(
        _proj_body,
        grid=grid,
        in_specs=[
            pl.BlockSpec((EMBED, B), lambda i: (0, 0)),
            pl.BlockSpec((EMBED, VB), lambda i: (0, i)),
            pl.BlockSpec((1, VB), lambda i: (0, i)),
        ],
        out_specs=pl.BlockSpec((VB, B), lambda i: (i, 0)),
        out_shape=jax.ShapeDtypeStruct((VOCAB, B), jnp.float32),
        scratch_shapes=[pltpu.VMEM((EMBED, B), jnp.float32)],
    )(xt, wt, b2d)


def kernel(context_words, emb, W, b):
    cwt = context_words.astype(jnp.int32).T
    xt = _poolt(cwt, emb.T)
    outt = _proj(xt, W.T, b.reshape(1, VOCAB))
    return outt.T
